# baseline (speedup 1.0000x reference)
import functools
import math

import jax
import jax.numpy as jnp
from jax import lax
from jax.experimental import pallas as pl
from jax.experimental.pallas import tpu as pltpu
from jax.experimental.pallas import tpu_sc as plsc

D_MODEL = 64
SCALE = math.sqrt(D_MODEL)
LANES = 16
NUM_CORES = 2
NUM_SUBCORES = 16
NUM_WORKERS = NUM_CORES * NUM_SUBCORES

RBLK = 4


@functools.lru_cache(maxsize=None)
def _build(S0, S1):
    rows_per_w = S0 // NUM_WORKERS
    nblocks = rows_per_w // RBLK
    splits = []
    off = 0
    while off < S1:
        n = min(128, S1 - off)
        splits.append((off, n))
        off += n

    mesh = plsc.VectorSubcoreMesh(
        core_axis_name="c", subcore_axis_name="s",
        num_cores=NUM_CORES, num_subcores=NUM_SUBCORES)

    @functools.partial(
        pl.kernel,
        mesh=mesh,
        out_type=jax.ShapeDtypeStruct((S0, S1, D_MODEL), jnp.float32),
        scratch_types=[
            pltpu.VMEM((RBLK, S1), jnp.int32),
            pltpu.VMEM((RBLK, S1, D_MODEL), jnp.float32),
            pltpu.SemaphoreType.DMA,
        ],
        compiler_params=pltpu.CompilerParams(use_tc_tiling_on_sc=False),
    )
    def emb(x_hbm, table_hbm, out_hbm, idx_v, rows_v, sem):
        wid = lax.axis_index("s") * NUM_CORES + lax.axis_index("c")
        row_base = wid * rows_per_w

        def block_body(bi, carry):
            r0 = row_base + bi * RBLK
            pltpu.sync_copy(x_hbm.at[pl.ds(r0, RBLK)], idx_v)
            copies = []
            for j in range(RBLK):
                for (o, n) in splits:
                    copies.append(pltpu.async_copy(
                        table_hbm.at[idx_v.at[j, pl.ds(o, n)]],
                        rows_v.at[j, pl.ds(o, n)],
                        sem))
            for c in copies:
                c.wait()

            for j in range(RBLK):
                def scale_row(r, c2, j=j):
                    for c in range(D_MODEL // LANES):
                        sl = pl.ds(c * LANES, LANES)
                        rows_v[j, r, sl] = rows_v[j, r, sl] * SCALE
                    return c2

                lax.fori_loop(0, S1, scale_row, 0, unroll=4)

            pltpu.sync_copy(rows_v, out_hbm.at[pl.ds(r0, RBLK)])
            return carry

        lax.fori_loop(0, nblocks, block_body, 0)

    return emb


def kernel(x, table):
    S0, S1 = x.shape
    return _build(S0, S1)(x.astype(jnp.int32), table)
